# baseline (device time: 15503 ns/iter reference)
import jax
import jax.numpy as jnp
from jax import lax
from jax.experimental import pallas as pl
from jax.experimental.pallas import tpu as pltpu

N_DEV = 8
N_STEPS = 3
BLK_ROWS = 8


def kernel(x):
    m, n = x.shape
    nblk = m // BLK_ROWS

    def body(x_ref, out_ref, incl_ref, excl_ref, recv_ref, send_sems, recv_sems):
        my = lax.axis_index("i")
        f32 = jnp.float32

        x2 = x_ref[...]

        xv3 = x2.reshape(nblk, BLK_ROWS, n)
        t = xv3[:, :4, :] * xv3[:, 4:, :]
        t = t[:, :2, :] * t[:, 2:, :]
        p8 = t[:, 0, :] * t[:, 1, :]

        b = p8
        d = 1
        while d < nblk:
            b = b * jnp.concatenate(
                [jnp.ones((d, n), f32), b[: nblk - d, :]], axis=0
            )
            d *= 2
        incl_ref[0:1, :] = b[nblk - 1 : nblk, :]
        excl_ref[0, :] = jnp.ones((n,), f32)

        row = lax.broadcasted_iota(jnp.int32, (m, 1), 0) % BLK_ROWS

        for s in range(N_STEPS):
            dd = 1 << s
            sends = my + dd < N_DEV
            recvs = my - dd >= 0
            copy = pltpu.make_async_remote_copy(
                src_ref=incl_ref,
                dst_ref=recv_ref.at[s],
                send_sem=send_sems.at[s],
                recv_sem=recv_sems.at[s],
                device_id=(jnp.minimum(my + dd, N_DEV - 1),),
                device_id_type=pl.DeviceIdType.MESH,
            )

            @pl.when(sends)
            def _():
                copy.start()

            shifted = jnp.concatenate(
                [jnp.ones((dd, n), f32), x2[: m - dd, :]], axis=0
            )
            x2 = x2 * jnp.where(row >= dd, shifted, f32(1.0))

            @pl.when(recvs)
            def _():
                copy.wait_recv()

            @pl.when(sends)
            def _():
                copy.wait_send()

            @pl.when(recvs)
            def _():
                r = recv_ref[s, 0, :]
                incl_ref[0, :] = incl_ref[0, :] * r
                excl_ref[0, :] = excl_ref[0, :] * r

        bex = (
            jnp.concatenate(
                [jnp.ones((1, n), f32), b[: nblk - 1, :]], axis=0
            )
            * excl_ref[0:1, :]
        )
        out_ref[...] = (
            x2.reshape(nblk, BLK_ROWS, n) * bex[:, None, :]
        ).reshape(m, n)

    return pl.pallas_call(
        body,
        out_shape=jax.ShapeDtypeStruct((m, n), jnp.float32),
        in_specs=[pl.BlockSpec(memory_space=pltpu.VMEM)],
        out_specs=pl.BlockSpec(memory_space=pltpu.VMEM),
        scratch_shapes=[
            pltpu.VMEM((1, n), jnp.float32),
            pltpu.VMEM((1, n), jnp.float32),
            pltpu.VMEM((N_STEPS, 1, n), jnp.float32),
            pltpu.SemaphoreType.DMA((N_STEPS,)),
            pltpu.SemaphoreType.DMA((N_STEPS,)),
        ],
    )(x)


# device time: 13584 ns/iter; 1.1413x vs baseline; 1.1413x over previous
import jax
import jax.numpy as jnp
from jax import lax
from jax.experimental import pallas as pl
from jax.experimental.pallas import tpu as pltpu

N_DEV = 8


def _hs_cumprod(x):
    m, n = x.shape
    d = 1
    while d < m:
        x = x * jnp.concatenate(
            [jnp.ones((d, n), x.dtype), x[: m - d, :]], axis=0
        )
        d *= 2
    return x


def _tree_prod(x):
    while x.shape[0] > 1:
        h = x.shape[0] // 2
        x = x[:h, :] * x[h:, :]
    return x


def kernel(x):
    m, n = x.shape

    def body(x_ref, out_ref, tot_ref, recv_buf, send_sems, recv_sems):
        my = lax.axis_index("i")
        f32 = jnp.float32
        xv = x_ref[...]

        tot_ref[0:1, :] = _tree_prod(xv)

        copies = []
        for j in range(N_DEV):
            c = pltpu.make_async_remote_copy(
                src_ref=tot_ref,
                dst_ref=recv_buf.at[pl.ds(my, 1), :],
                send_sem=send_sems.at[j],
                recv_sem=recv_sems.at[my],
                device_id=(j,),
                device_id_type=pl.DeviceIdType.MESH,
            )
            copies.append(c)

            @pl.when(my != j)
            def _(c=c):
                c.start()

        out_ref[...] = _hs_cumprod(xv)

        recv_buf[pl.ds(my, 1), :] = tot_ref[0:1, :]

        for j in range(N_DEV):
            rc = pltpu.make_async_remote_copy(
                src_ref=tot_ref,
                dst_ref=recv_buf.at[pl.ds(j, 1), :],
                send_sem=send_sems.at[j],
                recv_sem=recv_sems.at[j],
                device_id=(j,),
                device_id_type=pl.DeviceIdType.MESH,
            )

            @pl.when(my != j)
            def _(rc=rc):
                rc.wait_recv()

        for j in range(N_DEV):

            @pl.when(my != j)
            def _(c=copies[j]):
                c.wait_send()

        r8 = recv_buf[...]
        mask = lax.broadcasted_iota(jnp.int32, (N_DEV, 1), 0) < my
        excl = _tree_prod(jnp.where(mask, r8, jnp.ones((N_DEV, n), f32)))

        out_ref[...] = out_ref[...] * excl

    return pl.pallas_call(
        body,
        out_shape=jax.ShapeDtypeStruct((m, n), jnp.float32),
        in_specs=[pl.BlockSpec(memory_space=pltpu.VMEM)],
        out_specs=pl.BlockSpec(memory_space=pltpu.VMEM),
        scratch_shapes=[
            pltpu.VMEM((1, n), jnp.float32),
            pltpu.VMEM((N_DEV, n), jnp.float32),
            pltpu.SemaphoreType.DMA((N_DEV,)),
            pltpu.SemaphoreType.DMA((N_DEV,)),
        ],
    )(x)


# device time: 9416 ns/iter; 1.6465x vs baseline; 1.4427x over previous
import jax
import jax.numpy as jnp
from jax import lax
from jax.experimental import pallas as pl
from jax.experimental.pallas import tpu as pltpu

N_DEV = 8


def _hs_cumprod(x):
    m, n = x.shape
    d = 1
    while d < m:
        x = x * jnp.concatenate(
            [jnp.ones((d, n), x.dtype), x[: m - d, :]], axis=0
        )
        d *= 2
    return x


def _tree_prod(x):
    while x.shape[0] > 1:
        h = x.shape[0] // 2
        x = x[:h, :] * x[h:, :]
    return x


def kernel(x):
    m, n = x.shape

    def body(
        x_ref, out_ref, tot_ref, recv_buf, credit_sems, send_sems, recv_sems
    ):
        my = lax.axis_index("i")
        f32 = jnp.float32

        for k in range(N_DEV):

            @pl.when(k < my)
            def _(k=k):
                pl.semaphore_signal(
                    credit_sems.at[my],
                    inc=1,
                    device_id=(k,),
                    device_id_type=pl.DeviceIdType.MESH,
                )

        xv = x_ref[...]
        tot_ref[0:1, :] = _tree_prod(xv)

        copies = []
        for j in range(N_DEV):
            c = pltpu.make_async_remote_copy(
                src_ref=tot_ref,
                dst_ref=recv_buf.at[pl.ds(my, 1), :],
                send_sem=send_sems.at[j],
                recv_sem=recv_sems.at[my],
                device_id=(j,),
                device_id_type=pl.DeviceIdType.MESH,
            )
            copies.append(c)

            @pl.when(j > my)
            def _(c=c, j=j):
                pl.semaphore_wait(credit_sems.at[j], 1)
                c.start()

        out_ref[...] = _hs_cumprod(xv)

        for j in range(N_DEV):
            rc = pltpu.make_async_remote_copy(
                src_ref=tot_ref,
                dst_ref=recv_buf.at[pl.ds(j, 1), :],
                send_sem=send_sems.at[j],
                recv_sem=recv_sems.at[j],
                device_id=(j,),
                device_id_type=pl.DeviceIdType.MESH,
            )

            @pl.when(j < my)
            def _(rc=rc):
                rc.wait_recv()

        for j in range(N_DEV):

            @pl.when(j > my)
            def _(c=copies[j]):
                c.wait_send()

        recv_buf[pl.ds(my, 1), :] = jnp.ones((1, n), f32)
        r8 = recv_buf[...]
        mask = lax.broadcasted_iota(jnp.int32, (N_DEV, 1), 0) < my
        excl = _tree_prod(jnp.where(mask, r8, jnp.ones((N_DEV, n), f32)))

        out_ref[...] = out_ref[...] * excl

    return pl.pallas_call(
        body,
        out_shape=jax.ShapeDtypeStruct((m, n), jnp.float32),
        in_specs=[pl.BlockSpec(memory_space=pltpu.VMEM)],
        out_specs=pl.BlockSpec(memory_space=pltpu.VMEM),
        scratch_shapes=[
            pltpu.VMEM((1, n), jnp.float32),
            pltpu.VMEM((N_DEV, n), jnp.float32),
            pltpu.SemaphoreType.REGULAR((N_DEV,)),
            pltpu.SemaphoreType.DMA((N_DEV,)),
            pltpu.SemaphoreType.DMA((N_DEV,)),
        ],
        compiler_params=pltpu.CompilerParams(skip_device_barrier=True),
    )(x)
